# baseline (speedup 1.0000x reference)
import functools

import jax
import jax.numpy as jnp
from jax import lax
from jax.experimental import pallas as pl
from jax.experimental.pallas import tpu as pltpu
from jax.experimental.pallas import tpu_sc as plsc

LEN_VOCAB = 1000000
EMBED_DIM = 100
DPAD = 128
BATCH = 4096
HIST = 50

_info = plsc.get_sparse_core_info()
NC, NS = _info.num_cores, _info.num_subcores
NW = NC * NS
TOTAL = BATCH * HIST
PER_W = TOTAL // NW
CHUNK = 128
NCH = PER_W // CHUNK

PAD_ROWS = 4000
PAD_GRID = LEN_VOCAB // PAD_ROWS


def _pad_body(t_ref, o_ref):
    o_ref[:, :EMBED_DIM] = t_ref[...]
    o_ref[:, EMBED_DIM:] = jnp.zeros((PAD_ROWS, DPAD - EMBED_DIM), jnp.float32)


_pad_table = pl.pallas_call(
    _pad_body,
    grid=(PAD_GRID,),
    in_specs=[pl.BlockSpec((PAD_ROWS, EMBED_DIM), lambda i: (i, 0))],
    out_specs=pl.BlockSpec((PAD_ROWS, DPAD), lambda i: (i, 0)),
    out_shape=jax.ShapeDtypeStruct((LEN_VOCAB, DPAD), jnp.float32),
)


@functools.partial(
    pl.kernel,
    mesh=plsc.VectorSubcoreMesh(core_axis_name="c", subcore_axis_name="s"),
    out_type=jax.ShapeDtypeStruct((TOTAL, DPAD), jnp.float32),
    scratch_types=[
        pltpu.VMEM((PER_W,), jnp.int32),
        pltpu.VMEM((CHUNK, DPAD), jnp.float32),
        pltpu.VMEM((CHUNK, DPAD), jnp.float32),
        pltpu.SemaphoreType.DMA,
        pltpu.SemaphoreType.DMA,
    ],
)
def _gather_kernel(idx_hbm, table_hbm, out_hbm, idx_v, rows0, rows1, sem0, sem1):
    wid = lax.axis_index("s") * NC + lax.axis_index("c")
    base = wid * PER_W
    pltpu.sync_copy(idx_hbm.at[pl.ds(base, PER_W)], idx_v)

    bufs = (rows0, rows1)
    sems = (sem0, sem1)
    cps = [None, None]
    cps[0] = pltpu.async_copy(
        table_hbm.at[idx_v.at[pl.ds(0, CHUNK)]], bufs[0], sems[0])
    for j in range(NCH):
        nxt = j + 1
        if nxt < NCH:
            cps[nxt % 2] = pltpu.async_copy(
                table_hbm.at[idx_v.at[pl.ds(nxt * CHUNK, CHUNK)]],
                bufs[nxt % 2], sems[nxt % 2])
        cps[j % 2].wait()
        pltpu.sync_copy(bufs[j % 2], out_hbm.at[pl.ds(base + j * CHUNK, CHUNK)])


def kernel(x, table):
    idx = jnp.transpose(x).reshape(TOTAL).astype(jnp.int32)
    table_p = _pad_table(table)
    out = _gather_kernel(idx, table_p)
    return out[:, :EMBED_DIM].reshape(HIST, 1, BATCH, EMBED_DIM)


# speedup vs baseline: 1.0388x; 1.0388x over previous
import functools

import jax
import jax.numpy as jnp
from jax import lax
from jax.experimental import pallas as pl
from jax.experimental.pallas import tpu as pltpu
from jax.experimental.pallas import tpu_sc as plsc

LEN_VOCAB = 1000000
EMBED_DIM = 100
DPAD = 128
BATCH = 4096
HIST = 50

_info = plsc.get_sparse_core_info()
NC, NS = _info.num_cores, _info.num_subcores
NW = NC * NS
TOTAL = BATCH * HIST
PER_W = TOTAL // NW
CHUNK = 128
NCH = PER_W // CHUNK

PAD_ROWS = 20000
PAD_GRID = LEN_VOCAB // PAD_ROWS


def _pad_body(t_ref, o_ref):
    o_ref[:, :EMBED_DIM] = t_ref[...]
    o_ref[:, EMBED_DIM:] = jnp.zeros((PAD_ROWS, DPAD - EMBED_DIM), jnp.float32)


_pad_table = pl.pallas_call(
    _pad_body,
    grid=(PAD_GRID,),
    in_specs=[pl.BlockSpec((PAD_ROWS, EMBED_DIM), lambda i: (i, 0))],
    out_specs=pl.BlockSpec((PAD_ROWS, DPAD), lambda i: (i, 0)),
    out_shape=jax.ShapeDtypeStruct((LEN_VOCAB, DPAD), jnp.float32),
)


@functools.partial(
    pl.kernel,
    mesh=plsc.VectorSubcoreMesh(core_axis_name="c", subcore_axis_name="s"),
    out_type=jax.ShapeDtypeStruct((TOTAL, DPAD), jnp.float32),
    scratch_types=[
        pltpu.VMEM((PER_W,), jnp.int32),
        pltpu.VMEM((CHUNK, DPAD), jnp.float32),
        pltpu.VMEM((CHUNK, DPAD), jnp.float32),
        pltpu.SemaphoreType.DMA,
        pltpu.SemaphoreType.DMA,
    ],
)
def _gather_kernel(idx_hbm, table_hbm, out_hbm, idx_v, rows0, rows1, sem0, sem1):
    wid = lax.axis_index("s") * NC + lax.axis_index("c")
    base = wid * PER_W
    pltpu.sync_copy(idx_hbm.at[pl.ds(base, PER_W)], idx_v)

    bufs = (rows0, rows1)
    sems = (sem0, sem1)
    cps = [None, None]
    cps[0] = pltpu.async_copy(
        table_hbm.at[idx_v.at[pl.ds(0, CHUNK)]], bufs[0], sems[0])
    for j in range(NCH):
        nxt = j + 1
        if nxt < NCH:
            cps[nxt % 2] = pltpu.async_copy(
                table_hbm.at[idx_v.at[pl.ds(nxt * CHUNK, CHUNK)]],
                bufs[nxt % 2], sems[nxt % 2])
        cps[j % 2].wait()
        pltpu.sync_copy(bufs[j % 2], out_hbm.at[pl.ds(base + j * CHUNK, CHUNK)])


def kernel(x, table):
    idx = jnp.transpose(x).reshape(TOTAL).astype(jnp.int32)
    table_p = _pad_table(table)
    out = _gather_kernel(idx, table_p)
    return out[:, :EMBED_DIM].reshape(HIST, 1, BATCH, EMBED_DIM)


# speedup vs baseline: 1.0388x; 1.0000x over previous
import functools

import jax
import jax.numpy as jnp
from jax import lax
from jax.experimental import pallas as pl
from jax.experimental.pallas import tpu as pltpu
from jax.experimental.pallas import tpu_sc as plsc

LEN_VOCAB = 1000000
EMBED_DIM = 100
DPAD = 128
BATCH = 4096
HIST = 50

_info = plsc.get_sparse_core_info()
NC, NS = _info.num_cores, _info.num_subcores
NW = NC * NS
TOTAL = BATCH * HIST
PER_W = TOTAL // NW
CHUNK = 128
NCH = PER_W // CHUNK

PAD_ROWS = 20000
PAD_GRID = LEN_VOCAB // PAD_ROWS


def _pad_body(t_ref, o_ref):
    o_ref[...] = t_ref[...]


_pad_table = pl.pallas_call(
    _pad_body,
    grid=(PAD_GRID,),
    in_specs=[pl.BlockSpec((PAD_ROWS, DPAD), lambda i: (i, 0))],
    out_specs=pl.BlockSpec((PAD_ROWS, DPAD), lambda i: (i, 0)),
    out_shape=jax.ShapeDtypeStruct((LEN_VOCAB, DPAD), jnp.float32),
)


@functools.partial(
    pl.kernel,
    mesh=plsc.VectorSubcoreMesh(core_axis_name="c", subcore_axis_name="s"),
    out_type=jax.ShapeDtypeStruct((TOTAL, DPAD), jnp.float32),
    scratch_types=[
        pltpu.VMEM((PER_W,), jnp.int32),
        pltpu.VMEM((CHUNK, DPAD), jnp.float32),
        pltpu.VMEM((CHUNK, DPAD), jnp.float32),
        pltpu.SemaphoreType.DMA,
        pltpu.SemaphoreType.DMA,
    ],
)
def _gather_kernel(idx_hbm, table_hbm, out_hbm, idx_v, rows0, rows1, sem0, sem1):
    wid = lax.axis_index("s") * NC + lax.axis_index("c")
    base = wid * PER_W
    pltpu.sync_copy(idx_hbm.at[pl.ds(base, PER_W)], idx_v)

    bufs = (rows0, rows1)
    sems = (sem0, sem1)
    cps = [None, None]
    cps[0] = pltpu.async_copy(
        table_hbm.at[idx_v.at[pl.ds(0, CHUNK)]], bufs[0], sems[0])
    for j in range(NCH):
        nxt = j + 1
        if nxt < NCH:
            cps[nxt % 2] = pltpu.async_copy(
                table_hbm.at[idx_v.at[pl.ds(nxt * CHUNK, CHUNK)]],
                bufs[nxt % 2], sems[nxt % 2])
        cps[j % 2].wait()
        pltpu.sync_copy(bufs[j % 2], out_hbm.at[pl.ds(base + j * CHUNK, CHUNK)])


def kernel(x, table):
    idx = jnp.transpose(x).reshape(TOTAL).astype(jnp.int32)
    table_p = _pad_table(table)
    out = _gather_kernel(idx, table_p)
    return out[:, :EMBED_DIM].reshape(HIST, 1, BATCH, EMBED_DIM)
